# baseline (speedup 1.0000x reference)
import functools

import jax
import jax.numpy as jnp
from jax import lax
from jax.experimental import pallas as pl

_CLUSTERS = 64
_DIM = 32
_ITERATIONS = 2
_CHUNK = 512


def _dot(a, b):
    return lax.dot_general(a, b, (((1,), (0,)), ((), ())),
                           preferred_element_type=jnp.float32,
                           precision=lax.Precision.HIGHEST)


def _sub_min(x):
    return jnp.min(x, axis=0, keepdims=True)


def _assign_chunk(p, ct, cn_row, iota_k):
    p_t = p.T
    score_t = (cn_row - 2.0 * _dot(p, ct)).T
    m1 = _sub_min(score_t)
    a1 = _sub_min(jnp.where(score_t == m1, iota_k, _CLUSTERS))
    sc2 = jnp.where(iota_k == a1, jnp.inf, score_t)
    m2 = _sub_min(sc2)
    a2 = _sub_min(jnp.where(sc2 == m2, iota_k, _CLUSTERS))
    oh1 = jnp.where(iota_k == a1, 1.0, 0.0)
    oh2 = jnp.where(iota_k == a2, 1.0, 0.0)
    c1 = _dot(ct, oh1)
    c2 = _dot(ct, oh2)
    e1 = p_t - c1
    e2 = p_t - c2
    d1 = jnp.sum(e1 * e1, axis=0, keepdims=True)
    d2 = jnp.sum(e2 * e2, axis=0, keepdims=True)
    return jnp.where(d1 < d2, a1,
                     jnp.where(d2 < d1, a2, jnp.minimum(a1, a2)))


def _kmeans_body(points_ref, init_ct_ref, assign_ref, cent_ref):
    n = points_ref.shape[1]
    nchunks = n // _CHUNK
    iota_k = lax.broadcasted_iota(jnp.int32, (_CLUSTERS, _CHUNK), 0)
    ct = init_ct_ref[0]

    for _ in range(_ITERATIONS):
        cn_row = jnp.sum(ct * ct, axis=0, keepdims=True)

        def upd_step(i, carry):
            sums, counts = carry
            p = points_ref[0, pl.ds(i * _CHUNK, _CHUNK), :]
            a = _assign_chunk(p, ct, cn_row, iota_k)
            onehot = jnp.where(iota_k == a, 1.0, 0.0)
            sums = sums + _dot(onehot, p)
            counts = counts + jnp.sum(onehot, axis=1, keepdims=True)
            return sums, counts

        sums, counts = lax.fori_loop(
            0, nchunks, upd_step,
            (jnp.zeros((_CLUSTERS, _DIM), jnp.float32),
             jnp.zeros((_CLUSTERS, 1), jnp.float32)))
        c = sums / counts
        ct = c.T

    cn_row = jnp.sum(ct * ct, axis=0, keepdims=True)

    def assign_step(i, _):
        p = points_ref[0, pl.ds(i * _CHUNK, _CHUNK), :]
        a = _assign_chunk(p, ct, cn_row, iota_k)
        assign_ref[0, 0, pl.ds(i * _CHUNK, _CHUNK)] = a[0]
        return 0

    lax.fori_loop(0, nchunks, assign_step, 0)
    cent_ref[0] = c


@jax.jit
def kernel(points):
    b, n, dim = points.shape
    perm = jax.random.permutation(jax.random.key(42), n)
    init_ct = jnp.swapaxes(points[:, perm[:_CLUSTERS], :], 1, 2)

    assign, cent = pl.pallas_call(
        _kmeans_body,
        grid=(b,),
        in_specs=[
            pl.BlockSpec((1, n, dim), lambda i: (i, 0, 0)),
            pl.BlockSpec((1, dim, _CLUSTERS), lambda i: (i, 0, 0)),
        ],
        out_specs=[
            pl.BlockSpec((1, 1, n), lambda i: (i, 0, 0)),
            pl.BlockSpec((1, _CLUSTERS, dim), lambda i: (i, 0, 0)),
        ],
        out_shape=[
            jax.ShapeDtypeStruct((b, 1, n), jnp.int32),
            jax.ShapeDtypeStruct((b, _CLUSTERS, dim), jnp.float32),
        ],
    )(points, init_ct)
    return assign.reshape(b, n), cent


# speedup vs baseline: 1.1911x; 1.1911x over previous
import functools

import jax
import jax.numpy as jnp
from jax import lax
from jax.experimental import pallas as pl

_CLUSTERS = 64
_DIM = 32
_ITERATIONS = 2
_CHUNK = 512


def _dot(a, b, precision=lax.Precision.HIGHEST):
    return lax.dot_general(a, b, (((1,), (0,)), ((), ())),
                           preferred_element_type=jnp.float32,
                           precision=precision)


def _sub_min(x):
    return jnp.min(x, axis=0, keepdims=True)


def _assign_chunk(p, ct, cn_row, iota_k):
    p_t = p.T
    score_t = (cn_row - 2.0 * _dot(p, ct)).T
    m1 = _sub_min(score_t)
    a1 = _sub_min(jnp.where(score_t == m1, iota_k, _CLUSTERS))
    sc2 = jnp.where(iota_k == a1, jnp.inf, score_t)
    m2 = _sub_min(sc2)
    a2 = _sub_min(jnp.where(sc2 == m2, iota_k, _CLUSTERS))
    idx1 = jnp.broadcast_to(a1, (_DIM, a1.shape[1]))
    idx2 = jnp.broadcast_to(a2, (_DIM, a2.shape[1]))
    c1 = jnp.take_along_axis(ct, idx1, axis=1)
    c2 = jnp.take_along_axis(ct, idx2, axis=1)
    e1 = p_t - c1
    e2 = p_t - c2
    d1 = jnp.sum(e1 * e1, axis=0, keepdims=True)
    d2 = jnp.sum(e2 * e2, axis=0, keepdims=True)
    return jnp.where(d1 < d2, a1,
                     jnp.where(d2 < d1, a2, jnp.minimum(a1, a2)))


def _kmeans_body(points_ref, init_ct_ref, assign_ref, cent_ref):
    n = points_ref.shape[1]
    nchunks = n // _CHUNK
    iota_k = lax.broadcasted_iota(jnp.int32, (_CLUSTERS, _CHUNK), 0)
    ct = init_ct_ref[0]

    for _ in range(_ITERATIONS):
        cn_row = jnp.sum(ct * ct, axis=0, keepdims=True)

        def upd_step(i, carry):
            sums, counts = carry
            p = points_ref[0, pl.ds(i * _CHUNK, _CHUNK), :]
            a = _assign_chunk(p, ct, cn_row, iota_k)
            onehot = jnp.where(iota_k == a, 1.0, 0.0)
            sums = sums + _dot(onehot, p)
            counts = counts + jnp.sum(onehot, axis=1, keepdims=True)
            return sums, counts

        sums, counts = lax.fori_loop(
            0, nchunks, upd_step,
            (jnp.zeros((_CLUSTERS, _DIM), jnp.float32),
             jnp.zeros((_CLUSTERS, 1), jnp.float32)))
        c = sums / counts
        ct = c.T

    cn_row = jnp.sum(ct * ct, axis=0, keepdims=True)

    def assign_step(i, _):
        p = points_ref[0, pl.ds(i * _CHUNK, _CHUNK), :]
        a = _assign_chunk(p, ct, cn_row, iota_k)
        assign_ref[0, 0, pl.ds(i * _CHUNK, _CHUNK)] = a[0]
        return 0

    lax.fori_loop(0, nchunks, assign_step, 0)
    cent_ref[0] = c


@jax.jit
def kernel(points):
    b, n, dim = points.shape
    perm = jax.random.permutation(jax.random.key(42), n)
    init_ct = jnp.swapaxes(points[:, perm[:_CLUSTERS], :], 1, 2)

    assign, cent = pl.pallas_call(
        _kmeans_body,
        grid=(b,),
        in_specs=[
            pl.BlockSpec((1, n, dim), lambda i: (i, 0, 0)),
            pl.BlockSpec((1, dim, _CLUSTERS), lambda i: (i, 0, 0)),
        ],
        out_specs=[
            pl.BlockSpec((1, 1, n), lambda i: (i, 0, 0)),
            pl.BlockSpec((1, _CLUSTERS, dim), lambda i: (i, 0, 0)),
        ],
        out_shape=[
            jax.ShapeDtypeStruct((b, 1, n), jnp.int32),
            jax.ShapeDtypeStruct((b, _CLUSTERS, dim), jnp.float32),
        ],
    )(points, init_ct)
    return assign.reshape(b, n), cent
